# baseline (speedup 1.0000x reference)
import functools

import jax
import jax.numpy as jnp
from jax import lax
from jax.experimental import pallas as pl
from jax.experimental.pallas import tpu as pltpu
from jax.experimental.pallas import tpu_sc as plsc

ROWS = 128
COLS = 32768
LANES = 16
NUM_CORES = 2
NUM_SUBCORES = 16
NUM_WORKERS = NUM_CORES * NUM_SUBCORES
ROWS_PER_WORKER = ROWS // NUM_WORKERS
CHUNK = 4096
NUM_CHUNKS = COLS // CHUNK
MWORDS = CHUNK // 4
WORDS_PER_CHUNK = CHUNK // 64


def _sc_masked_cumsum(x_hbm, m_hbm, out_hbm,
                      xb0, xb1, mb0, mb1, sem_in0, sem_in1, sem_out):
    wid = lax.axis_index("s") * NUM_CORES + lax.axis_index("c")
    row0 = wid * ROWS_PER_WORKER
    xb = (xb0, xb1)
    mb = (mb0, mb1)
    sem_in = (sem_in0, sem_in1)

    def start_in(c, s):
        col = pl.ds(c * CHUNK, CHUNK)
        mcol = pl.ds(c * MWORDS, MWORDS)
        h = []
        for r in range(ROWS_PER_WORKER):
            h.append(pltpu.async_copy(x_hbm.at[row0 + r, col], xb[s].at[r],
                                      sem_in[s]))
            h.append(pltpu.async_copy(m_hbm.at[row0 + r, mcol], mb[s].at[r],
                                      sem_in[s]))
        return h

    def start_out(c, s):
        col = pl.ds(c * CHUNK, CHUNK)
        return [pltpu.async_copy(xb[s].at[r], out_hbm.at[row0 + r, col],
                                 sem_out)
                for r in range(ROWS_PER_WORKER)]

    carries = (jnp.float32(0.0),) * ROWS_PER_WORKER
    in_h = {0: start_in(0, 0)}
    out_h = {}
    for c in range(NUM_CHUNKS):
        s = c & 1
        if c + 1 < NUM_CHUNKS:
            if c - 1 >= 0:
                for h in out_h.pop(c - 1):
                    h.wait()
            in_h[c + 1] = start_in(c + 1, 1 - s)
        for h in in_h.pop(c):
            h.wait()

        xbuf, mbuf = xb[s], mb[s]

        def body(j, carry, xbuf=xbuf, mbuf=mbuf):
            out = []
            for r in range(ROWS_PER_WORKER):
                w = mbuf[r, pl.ds(j * LANES, LANES)]
                cr = carry[r]
                for k in range(4):
                    mf = ((w >> (8 * k)) & 1).astype(jnp.float32)
                    sl = (r, pl.ds(j * 64 + k * LANES, LANES))
                    v = xbuf[sl] * mf
                    sc = plsc.cumsum(v)
                    xbuf[sl] = sc + cr
                    cr = cr + sc[LANES - 1]
                out.append(cr)
            return tuple(out)

        carries = lax.fori_loop(0, WORDS_PER_CHUNK, body, carries, unroll=2)
        out_h[c] = start_out(c, s)
    for c in (NUM_CHUNKS - 2, NUM_CHUNKS - 1):
        for h in out_h.pop(c, ()):
            h.wait()


@jax.jit
def _masked_cumsum(x, mask):
    mm = mask.astype(jnp.int8).reshape(ROWS, COLS // 64, 4, LANES)
    mw = lax.bitcast_convert_type(mm.transpose(0, 1, 3, 2), jnp.int32)
    mw = mw.reshape(ROWS, COLS // 4)

    mesh = plsc.VectorSubcoreMesh(core_axis_name="c", subcore_axis_name="s")
    kern = functools.partial(
        pl.kernel,
        out_type=jax.ShapeDtypeStruct((ROWS, COLS), jnp.float32),
        mesh=mesh,
        scratch_types=[
            pltpu.VMEM((ROWS_PER_WORKER, CHUNK), jnp.float32),
            pltpu.VMEM((ROWS_PER_WORKER, CHUNK), jnp.float32),
            pltpu.VMEM((ROWS_PER_WORKER, MWORDS), jnp.int32),
            pltpu.VMEM((ROWS_PER_WORKER, MWORDS), jnp.int32),
            pltpu.SemaphoreType.DMA,
            pltpu.SemaphoreType.DMA,
            pltpu.SemaphoreType.DMA,
        ],
        compiler_params=pltpu.CompilerParams(needs_layout_passes=False),
    )(_sc_masked_cumsum)
    return kern(x, mw)


def kernel(x, mask):
    return _masked_cumsum(x, mask)


# speedup vs baseline: 1.1077x; 1.1077x over previous
import functools

import jax
import jax.numpy as jnp
from jax import lax
from jax.experimental import pallas as pl
from jax.experimental.pallas import tpu as pltpu
from jax.experimental.pallas import tpu_sc as plsc

ROWS = 128
COLS = 32768
LANES = 16
NUM_CORES = 2
NUM_SUBCORES = 16
NUM_WORKERS = NUM_CORES * NUM_SUBCORES
ROWS_PER_WORKER = ROWS // NUM_WORKERS
CHUNK = 4096
NUM_CHUNKS = COLS // CHUNK
MWORDS = CHUNK // 4
WORDS_PER_CHUNK = CHUNK // 64


def _sc_masked_cumsum(x_hbm, m_hbm, out_hbm,
                      xb0, xb1, mb0, mb1, sem_in0, sem_in1, sem_out):
    wid = lax.axis_index("s") * NUM_CORES + lax.axis_index("c")
    row0 = wid * ROWS_PER_WORKER
    xb = (xb0, xb1)
    mb = (mb0, mb1)
    sem_in = (sem_in0, sem_in1)

    def start_in(c, s):
        col = pl.ds(c * CHUNK, CHUNK)
        mcol = pl.ds(c * MWORDS, MWORDS)
        h = []
        for r in range(ROWS_PER_WORKER):
            h.append(pltpu.async_copy(x_hbm.at[row0 + r, col], xb[s].at[r],
                                      sem_in[s]))
            h.append(pltpu.async_copy(m_hbm.at[row0 + r, mcol], mb[s].at[r],
                                      sem_in[s]))
        return h

    def start_out(c, s):
        col = pl.ds(c * CHUNK, CHUNK)
        return [pltpu.async_copy(xb[s].at[r], out_hbm.at[row0 + r, col],
                                 sem_out)
                for r in range(ROWS_PER_WORKER)]

    carries = (jnp.float32(0.0),) * ROWS_PER_WORKER
    in_h = {0: start_in(0, 0)}
    out_h = {}
    for c in range(NUM_CHUNKS):
        s = c & 1
        if c + 1 < NUM_CHUNKS:
            if c - 1 >= 0:
                for h in out_h.pop(c - 1):
                    h.wait()
            in_h[c + 1] = start_in(c + 1, 1 - s)
        for h in in_h.pop(c):
            h.wait()

        xbuf, mbuf = xb[s], mb[s]

        def body(j, carry, xbuf=xbuf, mbuf=mbuf):
            out = []
            for r in range(ROWS_PER_WORKER):
                w = mbuf[r, pl.ds(j * LANES, LANES)]
                cr = carry[r]
                for k in range(4):
                    mf = ((w >> (8 * k)) & 1).astype(jnp.float32)
                    sl = (r, pl.ds(j * 64 + k * LANES, LANES))
                    v = xbuf[sl] * mf
                    sc = plsc.cumsum(v)
                    xbuf[sl] = sc + cr
                    cr = cr + sc[LANES - 1]
                out.append(cr)
            return tuple(out)

        carries = lax.fori_loop(0, WORDS_PER_CHUNK, body, carries, unroll=2)
        out_h[c] = start_out(c, s)
    for c in (NUM_CHUNKS - 2, NUM_CHUNKS - 1):
        for h in out_h.pop(c, ()):
            h.wait()


@jax.jit
def _masked_cumsum(x, mask):
    mi = mask.astype(jnp.int32).reshape(ROWS, COLS // 64, 4, LANES)
    w8 = jnp.array([1, 1 << 8, 1 << 16, 1 << 24], jnp.int32)
    mw = (mi * w8[None, None, :, None]).sum(axis=2).reshape(ROWS, COLS // 4)

    mesh = plsc.VectorSubcoreMesh(core_axis_name="c", subcore_axis_name="s")
    kern = functools.partial(
        pl.kernel,
        out_type=jax.ShapeDtypeStruct((ROWS, COLS), jnp.float32),
        mesh=mesh,
        scratch_types=[
            pltpu.VMEM((ROWS_PER_WORKER, CHUNK), jnp.float32),
            pltpu.VMEM((ROWS_PER_WORKER, CHUNK), jnp.float32),
            pltpu.VMEM((ROWS_PER_WORKER, MWORDS), jnp.int32),
            pltpu.VMEM((ROWS_PER_WORKER, MWORDS), jnp.int32),
            pltpu.SemaphoreType.DMA,
            pltpu.SemaphoreType.DMA,
            pltpu.SemaphoreType.DMA,
        ],
        compiler_params=pltpu.CompilerParams(needs_layout_passes=False),
    )(_sc_masked_cumsum)
    return kern(x, mw)


def kernel(x, mask):
    return _masked_cumsum(x, mask)


# speedup vs baseline: 1.1547x; 1.0424x over previous
import functools

import jax
import jax.numpy as jnp
from jax import lax
from jax.experimental import pallas as pl
from jax.experimental.pallas import tpu as pltpu
from jax.experimental.pallas import tpu_sc as plsc

ROWS = 128
COLS = 32768
LANES = 16
NUM_CORES = 2
NUM_SUBCORES = 16
NUM_WORKERS = NUM_CORES * NUM_SUBCORES
ROWS_PER_WORKER = ROWS // NUM_WORKERS
CHUNK = 4096
NUM_CHUNKS = COLS // CHUNK
VREGS_PER_CHUNK = CHUNK // LANES


def _sc_masked_cumsum(x_hbm, m_hbm, out_hbm,
                      xb0, xb1, mb0, mb1, sem_in0, sem_in1, sem_out):
    wid = lax.axis_index("s") * NUM_CORES + lax.axis_index("c")
    row0 = wid * ROWS_PER_WORKER
    xb = (xb0, xb1)
    mb = (mb0, mb1)
    sem_in = (sem_in0, sem_in1)

    def start_in(c, s):
        col = pl.ds(c * CHUNK, CHUNK)
        h = []
        for r in range(ROWS_PER_WORKER):
            h.append(pltpu.async_copy(x_hbm.at[row0 + r, col], xb[s].at[r],
                                      sem_in[s]))
            h.append(pltpu.async_copy(m_hbm.at[row0 + r, col], mb[s].at[r],
                                      sem_in[s]))
        return h

    def start_out(c, s):
        col = pl.ds(c * CHUNK, CHUNK)
        return [pltpu.async_copy(xb[s].at[r], out_hbm.at[row0 + r, col],
                                 sem_out)
                for r in range(ROWS_PER_WORKER)]

    carries = (jnp.float32(0.0),) * ROWS_PER_WORKER
    in_h = {0: start_in(0, 0)}
    out_h = {}
    for c in range(NUM_CHUNKS):
        s = c & 1
        if c + 1 < NUM_CHUNKS:
            if c - 1 >= 0:
                for h in out_h.pop(c - 1):
                    h.wait()
            in_h[c + 1] = start_in(c + 1, 1 - s)
        for h in in_h.pop(c):
            h.wait()

        xbuf, mbuf = xb[s], mb[s]

        def body(j, carry, xbuf=xbuf, mbuf=mbuf):
            base = j * LANES
            out = []
            for r in range(ROWS_PER_WORKER):
                sl = (r, pl.ds(base, LANES))
                v = xbuf[sl] * mbuf[sl]
                sc = plsc.cumsum(v)
                xbuf[sl] = sc + carry[r]
                out.append(carry[r] + sc[LANES - 1])
            return tuple(out)

        carries = lax.fori_loop(0, VREGS_PER_CHUNK, body, carries, unroll=4)
        out_h[c] = start_out(c, s)
    for c in (NUM_CHUNKS - 2, NUM_CHUNKS - 1):
        for h in out_h.pop(c, ()):
            h.wait()


@jax.jit
def _masked_cumsum(x, mask_f32):
    mesh = plsc.VectorSubcoreMesh(core_axis_name="c", subcore_axis_name="s")
    kern = functools.partial(
        pl.kernel,
        out_type=jax.ShapeDtypeStruct((ROWS, COLS), jnp.float32),
        mesh=mesh,
        scratch_types=[
            pltpu.VMEM((ROWS_PER_WORKER, CHUNK), jnp.float32),
            pltpu.VMEM((ROWS_PER_WORKER, CHUNK), jnp.float32),
            pltpu.VMEM((ROWS_PER_WORKER, CHUNK), jnp.float32),
            pltpu.VMEM((ROWS_PER_WORKER, CHUNK), jnp.float32),
            pltpu.SemaphoreType.DMA,
            pltpu.SemaphoreType.DMA,
            pltpu.SemaphoreType.DMA,
        ],
        compiler_params=pltpu.CompilerParams(needs_layout_passes=False),
    )(_sc_masked_cumsum)
    return kern(x, mask_f32)


def kernel(x, mask):
    return _masked_cumsum(x, mask.astype(jnp.float32))


# speedup vs baseline: 1.4244x; 1.2336x over previous
import functools

import jax
import jax.numpy as jnp
from jax import lax
from jax.experimental import pallas as pl
from jax.experimental.pallas import tpu as pltpu
from jax.experimental.pallas import tpu_sc as plsc

ROWS = 128
COLS = 32768
LANES = 16
NUM_CORES = 2
NUM_SUBCORES = 16
NUM_WORKERS = NUM_CORES * NUM_SUBCORES
ROWS_PER_WORKER = ROWS // NUM_WORKERS
CHUNK = 4096
NUM_CHUNKS = COLS // CHUNK
VREGS_PER_CHUNK = CHUNK // LANES


def _sc_masked_cumsum(x_hbm, m_hbm, out_hbm,
                      xb0, xb1, mb0, mb1, sem_in0, sem_in1, sem_out):
    wid = lax.axis_index("s") * NUM_CORES + lax.axis_index("c")
    row0 = wid * ROWS_PER_WORKER
    xb = (xb0, xb1)
    mb = (mb0, mb1)
    sem_in = (sem_in0, sem_in1)

    def start_in(c, s):
        col = pl.ds(c * CHUNK, CHUNK)
        h = []
        for r in range(ROWS_PER_WORKER):
            h.append(pltpu.async_copy(x_hbm.at[row0 + r, col], xb[s].at[r],
                                      sem_in[s]))
            h.append(pltpu.async_copy(m_hbm.at[row0 + r, col], mb[s].at[r],
                                      sem_in[s]))
        return h

    def start_out(c, s):
        col = pl.ds(c * CHUNK, CHUNK)
        return [pltpu.async_copy(xb[s].at[r], out_hbm.at[row0 + r, col],
                                 sem_out)
                for r in range(ROWS_PER_WORKER)]

    carries = (jnp.zeros((LANES,), jnp.float32),) * ROWS_PER_WORKER
    in_h = {0: start_in(0, 0)}
    out_h = {}
    for c in range(NUM_CHUNKS):
        s = c & 1
        if c + 1 < NUM_CHUNKS:
            if c - 1 >= 0:
                for h in out_h.pop(c - 1):
                    h.wait()
            in_h[c + 1] = start_in(c + 1, 1 - s)
        for h in in_h.pop(c):
            h.wait()

        xbuf, mbuf = xb[s], mb[s]

        def body(j, carry, xbuf=xbuf, mbuf=mbuf):
            base = j * LANES
            last = jnp.full((LANES,), LANES - 1, jnp.int32)
            out = []
            for r in range(ROWS_PER_WORKER):
                sl = (r, pl.ds(base, LANES))
                v = xbuf[sl] * mbuf[sl]
                sc = plsc.cumsum(v)
                ov = sc + carry[r]
                xbuf[sl] = ov
                out.append(jnp.take_along_axis(
                    ov, last, axis=0, mode="promise_in_bounds"))
            return tuple(out)

        carries = lax.fori_loop(0, VREGS_PER_CHUNK, body, carries, unroll=2)
        out_h[c] = start_out(c, s)
    for c in (NUM_CHUNKS - 2, NUM_CHUNKS - 1):
        for h in out_h.pop(c, ()):
            h.wait()


@jax.jit
def _masked_cumsum(x, mask_f32):
    mesh = plsc.VectorSubcoreMesh(core_axis_name="c", subcore_axis_name="s")
    kern = functools.partial(
        pl.kernel,
        out_type=jax.ShapeDtypeStruct((ROWS, COLS), jnp.float32),
        mesh=mesh,
        scratch_types=[
            pltpu.VMEM((ROWS_PER_WORKER, CHUNK), jnp.float32),
            pltpu.VMEM((ROWS_PER_WORKER, CHUNK), jnp.float32),
            pltpu.VMEM((ROWS_PER_WORKER, CHUNK), jnp.float32),
            pltpu.VMEM((ROWS_PER_WORKER, CHUNK), jnp.float32),
            pltpu.SemaphoreType.DMA,
            pltpu.SemaphoreType.DMA,
            pltpu.SemaphoreType.DMA,
        ],
        compiler_params=pltpu.CompilerParams(needs_layout_passes=False),
    )(_sc_masked_cumsum)
    return kern(x, mask_f32)


def kernel(x, mask):
    return _masked_cumsum(x, mask.astype(jnp.float32))


# speedup vs baseline: 2.5084x; 1.7610x over previous
import functools

import jax
import jax.numpy as jnp
from jax import lax
from jax.experimental import pallas as pl
from jax.experimental.pallas import tpu as pltpu
from jax.experimental.pallas import tpu_sc as plsc

ROWS = 128
COLS = 32768
LANES = 16
NUM_CORES = 2
NUM_SUBCORES = 16
NUM_WORKERS = NUM_CORES * NUM_SUBCORES
ROWS_PER_WORKER = ROWS // NUM_WORKERS
CHUNK = 4096
NUM_CHUNKS = COLS // CHUNK
VREGS_PER_CHUNK = CHUNK // LANES


def _sc_masked_cumsum(x_hbm, m_hbm, out_hbm,
                      xb0, xb1, mb0, mb1, sem_in0, sem_in1, sem_out):
    wid = lax.axis_index("s") * NUM_CORES + lax.axis_index("c")
    row0 = wid * ROWS_PER_WORKER
    xb = (xb0, xb1)
    mb = (mb0, mb1)
    sem_in = (sem_in0, sem_in1)

    def start_in(c, s):
        col = pl.ds(c * CHUNK, CHUNK)
        h = []
        for r in range(ROWS_PER_WORKER):
            h.append(pltpu.async_copy(x_hbm.at[row0 + r, col], xb[s].at[r],
                                      sem_in[s]))
            h.append(pltpu.async_copy(m_hbm.at[row0 + r, col], mb[s].at[r],
                                      sem_in[s]))
        return h

    def start_out(c, s):
        col = pl.ds(c * CHUNK, CHUNK)
        return [pltpu.async_copy(xb[s].at[r], out_hbm.at[row0 + r, col],
                                 sem_out)
                for r in range(ROWS_PER_WORKER)]

    carries = (jnp.zeros((LANES,), jnp.float32),) * ROWS_PER_WORKER
    in_h = {0: start_in(0, 0)}
    out_h = {}
    for c in range(NUM_CHUNKS):
        s = c & 1
        if c + 1 < NUM_CHUNKS:
            if c - 1 >= 0:
                for h in out_h.pop(c - 1):
                    h.wait()
            in_h[c + 1] = start_in(c + 1, 1 - s)
        for h in in_h.pop(c):
            h.wait()

        xbuf, mbuf = xb[s], mb[s]

        @plsc.parallel_loop(0, VREGS_PER_CHUNK, 1, unroll=2, carry=carries)
        def body(j, carry, xbuf=xbuf, mbuf=mbuf):
            base = j * LANES
            last = jnp.full((LANES,), LANES - 1, jnp.int32)
            out = []
            for r in range(ROWS_PER_WORKER):
                sl = (r, pl.ds(base, LANES))
                v = xbuf[sl] * mbuf[sl]
                sc = plsc.cumsum(v)
                ov = sc + carry[r]
                xbuf[sl] = ov
                out.append(jnp.take_along_axis(
                    ov, last, axis=0, mode="promise_in_bounds"))
            return tuple(out)

        carries = body
        out_h[c] = start_out(c, s)
    for c in (NUM_CHUNKS - 2, NUM_CHUNKS - 1):
        for h in out_h.pop(c, ()):
            h.wait()


@jax.jit
def _masked_cumsum(x, mask_f32):
    mesh = plsc.VectorSubcoreMesh(core_axis_name="c", subcore_axis_name="s")
    kern = functools.partial(
        pl.kernel,
        out_type=jax.ShapeDtypeStruct((ROWS, COLS), jnp.float32),
        mesh=mesh,
        scratch_types=[
            pltpu.VMEM((ROWS_PER_WORKER, CHUNK), jnp.float32),
            pltpu.VMEM((ROWS_PER_WORKER, CHUNK), jnp.float32),
            pltpu.VMEM((ROWS_PER_WORKER, CHUNK), jnp.float32),
            pltpu.VMEM((ROWS_PER_WORKER, CHUNK), jnp.float32),
            pltpu.SemaphoreType.DMA,
            pltpu.SemaphoreType.DMA,
            pltpu.SemaphoreType.DMA,
        ],
        compiler_params=pltpu.CompilerParams(needs_layout_passes=False),
    )(_sc_masked_cumsum)
    return kern(x, mask_f32)


def kernel(x, mask):
    return _masked_cumsum(x, mask.astype(jnp.float32))
